# Initial kernel scaffold; baseline (speedup 1.0000x reference)
import jax
import jax.numpy as jnp
from jax.experimental import pallas as pl


def kernel(features, edge_index, W1, b1, W2, b2, W0, b0, Wf1, bf1, Wf2, bf2):
    raise NotImplementedError("write your pallas kernel here")


# speedup vs baseline: 7.9170x; 7.9170x over previous
import functools

import jax
import jax.numpy as jnp
from jax import lax
from jax.experimental import pallas as pl
from jax.experimental.pallas import tpu as pltpu
from jax.experimental.pallas import tpu_sc as plsc

N = 10000
E = 160000
NP = 10240
D = 256
DH = 128
NC = 2
NS = 16
EPC = E // NC
EPT_DEG = EPC // NS
DEG_CH = 40
EPT = E // NS
AGG_CH = 80
RPT = 632
ZPT = NP // NS
BM = 1000

_sc_mesh = plsc.VectorSubcoreMesh(core_axis_name="c", subcore_axis_name="s")


@functools.partial(
    pl.kernel,
    out_type=(jax.ShapeDtypeStruct((NP,), jnp.float32),
              jax.ShapeDtypeStruct((NP,), jnp.float32)),
    mesh=_sc_mesh,
    scratch_types=[
        pltpu.VMEM((DEG_CH,), jnp.int32),
        pltpu.VMEM((DEG_CH,), jnp.float32),
        pltpu.VMEM((ZPT,), jnp.float32),
        pltpu.VMEM_SHARED((NP,), jnp.float32),
    ],
)
def _deg_kernel(dst_hbm, deg0_hbm, deg1_hbm, idx_v, ones_v, z_v, deg_sh):
    c = lax.axis_index("c")
    s = lax.axis_index("s")
    one16 = jnp.ones((16,), jnp.float32)
    ones_v[pl.ds(0, 16)] = one16
    ones_v[pl.ds(16, 16)] = one16
    ones_v[pl.ds(24, 16)] = one16
    zero16 = jnp.zeros((16,), jnp.float32)

    def zfill(i, carry):
        z_v[pl.ds(i * 16, 16)] = zero16
        return carry

    lax.fori_loop(0, ZPT // 16, zfill, 0)
    pltpu.sync_copy(z_v, deg_sh.at[pl.ds(s * ZPT, ZPT)])
    plsc.subcore_barrier()

    base = c * EPC + s * EPT_DEG

    def body(i, carry):
        pltpu.sync_copy(dst_hbm.at[pl.ds(base + i * DEG_CH, DEG_CH)], idx_v)
        pltpu.sync_copy(ones_v, deg_sh.at[idx_v], add=True)
        return carry

    lax.fori_loop(0, EPT_DEG // DEG_CH, body, 0)
    plsc.subcore_barrier()

    @pl.when(c == 0)
    def _():
        pltpu.sync_copy(deg_sh.at[pl.ds(s * ZPT, ZPT)],
                        deg0_hbm.at[pl.ds(s * ZPT, ZPT)])

    @pl.when(c == 1)
    def _():
        pltpu.sync_copy(deg_sh.at[pl.ds(s * ZPT, ZPT)],
                        deg1_hbm.at[pl.ds(s * ZPT, ZPT)])


@functools.partial(
    pl.kernel,
    out_type=(jax.ShapeDtypeStruct((N, DH), jnp.float32),
              jax.ShapeDtypeStruct((N, DH), jnp.float32)),
    mesh=_sc_mesh,
    scratch_types=[
        pltpu.VMEM((AGG_CH,), jnp.int32),
        pltpu.VMEM((AGG_CH,), jnp.int32),
        pltpu.VMEM((AGG_CH, DH), jnp.float32),
        pltpu.VMEM_SHARED((N, DH), jnp.float32),
        pltpu.SemaphoreType.DMA,
    ],
)
def _agg_kernel(src_hbm, dst_hbm, hs0_hbm, hs1_hbm, out0_hbm, out1_hbm,
                srcv, dstv, rows_v, acc_sh, sem):
    c = lax.axis_index("c")
    s = lax.axis_index("s")

    def run(hs_hbm, out_hbm):
        @pl.when(s < NS - 1)
        def _():
            pltpu.sync_copy(hs_hbm.at[pl.ds(s * RPT, RPT)],
                            acc_sh.at[pl.ds(s * RPT, RPT)])

        @pl.when(s == NS - 1)
        def _():
            pltpu.sync_copy(hs_hbm.at[pl.ds((NS - 1) * RPT, N - (NS - 1) * RPT)],
                            acc_sh.at[pl.ds((NS - 1) * RPT, N - (NS - 1) * RPT)])

        plsc.subcore_barrier()

        def body(i, carry):
            off = s * EPT + i * AGG_CH
            pltpu.sync_copy(src_hbm.at[pl.ds(off, AGG_CH)], srcv)
            pltpu.sync_copy(dst_hbm.at[pl.ds(off, AGG_CH)], dstv)
            pltpu.async_copy(hs_hbm.at[srcv], rows_v, sem).wait()
            pltpu.sync_copy(rows_v, acc_sh.at[dstv], add=True)
            return carry

        lax.fori_loop(0, EPT // AGG_CH, body, 0)
        plsc.subcore_barrier()

        @pl.when(s < NS - 1)
        def _():
            pltpu.sync_copy(acc_sh.at[pl.ds(s * RPT, RPT)],
                            out_hbm.at[pl.ds(s * RPT, RPT)])

        @pl.when(s == NS - 1)
        def _():
            pltpu.sync_copy(acc_sh.at[pl.ds((NS - 1) * RPT, N - (NS - 1) * RPT)],
                            out_hbm.at[pl.ds((NS - 1) * RPT, N - (NS - 1) * RPT)])

    @pl.when(c == 0)
    def _():
        run(hs0_hbm, out0_hbm)

    @pl.when(c == 1)
    def _():
        run(hs1_hbm, out1_hbm)


def _mm1_body(feat_ref, w_ref, d0_ref, d1_ref, hs0_ref, hs1_ref, dinv_ref):
    dinv = lax.rsqrt(d0_ref[...] + d1_ref[...] + 1.0)
    h = jnp.dot(feat_ref[...], w_ref[...],
                preferred_element_type=jnp.float32)
    hs = h * dinv
    hs0_ref[...] = hs[:, :DH]
    hs1_ref[...] = hs[:, DH:]
    dinv_ref[...] = dinv


_mm1 = pl.pallas_call(
    _mm1_body,
    grid=(N // BM,),
    in_specs=[
        pl.BlockSpec((BM, D), lambda i: (i, 0)),
        pl.BlockSpec((D, D), lambda i: (0, 0)),
        pl.BlockSpec((BM, 1), lambda i: (i, 0)),
        pl.BlockSpec((BM, 1), lambda i: (i, 0)),
    ],
    out_specs=[
        pl.BlockSpec((BM, DH), lambda i: (i, 0)),
        pl.BlockSpec((BM, DH), lambda i: (i, 0)),
        pl.BlockSpec((BM, 1), lambda i: (i, 0)),
    ],
    out_shape=[
        jax.ShapeDtypeStruct((N, DH), jnp.float32),
        jax.ShapeDtypeStruct((N, DH), jnp.float32),
        jax.ShapeDtypeStruct((N, 1), jnp.float32),
    ],
)


def _mm2_body(a0_ref, a1_ref, dinv_ref, b_ref, w_ref, o0_ref, o1_ref):
    dinv = dinv_ref[...]
    x = jnp.concatenate([a0_ref[...], a1_ref[...]], axis=1) * dinv + b_ref[...]
    x = jnp.maximum(x, 0.0)
    h = jnp.dot(x, w_ref[...], preferred_element_type=jnp.float32)
    hs = h * dinv
    o0_ref[...] = hs[:, :DH]
    o1_ref[...] = hs[:, DH:]


_mm2 = pl.pallas_call(
    _mm2_body,
    grid=(N // BM,),
    in_specs=[
        pl.BlockSpec((BM, DH), lambda i: (i, 0)),
        pl.BlockSpec((BM, DH), lambda i: (i, 0)),
        pl.BlockSpec((BM, 1), lambda i: (i, 0)),
        pl.BlockSpec((1, D), lambda i: (0, 0)),
        pl.BlockSpec((D, D), lambda i: (0, 0)),
    ],
    out_specs=[
        pl.BlockSpec((BM, DH), lambda i: (i, 0)),
        pl.BlockSpec((BM, DH), lambda i: (i, 0)),
    ],
    out_shape=[
        jax.ShapeDtypeStruct((N, DH), jnp.float32),
        jax.ShapeDtypeStruct((N, DH), jnp.float32),
    ],
)


def _tail_body(a0_ref, a1_ref, dinv_ref, b2_ref, w0_ref, b0_ref,
               wf1_ref, bf1_ref, wf2_ref, bf2_ref,
               emb_ref, log_ref, gmax_ref):
    i = pl.program_id(0)
    x = jnp.concatenate([a0_ref[...], a1_ref[...]], axis=1) * dinv_ref[...] \
        + b2_ref[...]
    x = jnp.maximum(x, 0.0)
    m = jnp.max(x, axis=0, keepdims=True)

    @pl.when(i == 0)
    def _():
        gmax_ref[...] = m

    @pl.when(i > 0)
    def _():
        gmax_ref[...] = jnp.maximum(gmax_ref[...], m)

    @pl.when(i == pl.num_programs(0) - 1)
    def _():
        g = gmax_ref[...]
        emb = jnp.dot(g, w0_ref[...], preferred_element_type=jnp.float32) \
            + b0_ref[...]
        emb_ref[...] = emb
        hh = jnp.maximum(
            jnp.dot(emb, wf1_ref[...], preferred_element_type=jnp.float32)
            + bf1_ref[...], 0.0)
        log_ref[...] = jnp.dot(hh, wf2_ref[...],
                               preferred_element_type=jnp.float32) + bf2_ref[...]


_tail = pl.pallas_call(
    _tail_body,
    grid=(N // BM,),
    in_specs=[
        pl.BlockSpec((BM, DH), lambda i: (i, 0)),
        pl.BlockSpec((BM, DH), lambda i: (i, 0)),
        pl.BlockSpec((BM, 1), lambda i: (i, 0)),
        pl.BlockSpec((1, D), lambda i: (0, 0)),
        pl.BlockSpec((D, DH), lambda i: (0, 0)),
        pl.BlockSpec((1, DH), lambda i: (0, 0)),
        pl.BlockSpec((DH, D), lambda i: (0, 0)),
        pl.BlockSpec((1, D), lambda i: (0, 0)),
        pl.BlockSpec((D, 10), lambda i: (0, 0)),
        pl.BlockSpec((1, 10), lambda i: (0, 0)),
    ],
    out_specs=[
        pl.BlockSpec((1, DH), lambda i: (0, 0)),
        pl.BlockSpec((1, 10), lambda i: (0, 0)),
    ],
    out_shape=[
        jax.ShapeDtypeStruct((1, DH), jnp.float32),
        jax.ShapeDtypeStruct((1, 10), jnp.float32),
    ],
    scratch_shapes=[pltpu.VMEM((1, D), jnp.float32)],
)


def kernel(features, edge_index, W1, b1, W2, b2, W0, b0, Wf1, bf1, Wf2, bf2):
    src = edge_index[0]
    dst = edge_index[1]

    d0p, d1p = _deg_kernel(dst)
    d0 = d0p[:N].reshape(N, 1)
    d1 = d1p[:N].reshape(N, 1)

    hs0, hs1, dinv = _mm1(features, W1, d0, d1)
    a0, a1 = _agg_kernel(src, dst, hs0, hs1)
    g0, g1 = _mm2(a0, a1, dinv, b1.reshape(1, D), W2)
    c0, c1 = _agg_kernel(src, dst, g0, g1)
    emb, logits = _tail(c0, c1, dinv, b2.reshape(1, D),
                        W0, b0.reshape(1, DH),
                        Wf1, bf1.reshape(1, D),
                        Wf2, bf2.reshape(1, 10))
    return emb.reshape(DH), logits.reshape(10)


# speedup vs baseline: 14.8068x; 1.8703x over previous
import functools

import jax
import jax.numpy as jnp
from jax import lax
from jax.experimental import pallas as pl
from jax.experimental.pallas import tpu as pltpu
from jax.experimental.pallas import tpu_sc as plsc

N = 10000
E = 160000
NP = 10240
D = 256
DH = 128
NC = 2
NS = 16
EPC = E // NC
EPT_DEG = EPC // NS
DEG_CH = 40
EPT = E // NS
AGG_CH = 40
AGG_NCH = EPT // AGG_CH
RPT = 632
ZPT = NP // NS
BM = 1000

_sc_mesh = plsc.VectorSubcoreMesh(core_axis_name="c", subcore_axis_name="s")


@functools.partial(
    pl.kernel,
    out_type=(jax.ShapeDtypeStruct((NP,), jnp.float32),
              jax.ShapeDtypeStruct((NP,), jnp.float32)),
    mesh=_sc_mesh,
    scratch_types=[
        pltpu.VMEM((EPT_DEG,), jnp.int32),
        pltpu.VMEM((DEG_CH,), jnp.float32),
        pltpu.VMEM((ZPT,), jnp.float32),
        pltpu.VMEM_SHARED((NP,), jnp.float32),
    ],
)
def _deg_kernel(dst_hbm, deg0_hbm, deg1_hbm, idx_v, ones_v, z_v, deg_sh):
    c = lax.axis_index("c")
    s = lax.axis_index("s")
    one16 = jnp.ones((16,), jnp.float32)
    ones_v[pl.ds(0, 16)] = one16
    ones_v[pl.ds(16, 16)] = one16
    ones_v[pl.ds(24, 16)] = one16
    zero16 = jnp.zeros((16,), jnp.float32)

    def zfill(i, carry):
        z_v[pl.ds(i * 16, 16)] = zero16
        return carry

    lax.fori_loop(0, ZPT // 16, zfill, 0)
    pltpu.sync_copy(z_v, deg_sh.at[pl.ds(s * ZPT, ZPT)])
    plsc.subcore_barrier()

    base = c * EPC + s * EPT_DEG
    pltpu.sync_copy(dst_hbm.at[pl.ds(base, EPT_DEG)], idx_v)

    def body(i, carry):
        pltpu.sync_copy(ones_v,
                        deg_sh.at[idx_v.at[pl.ds(i * DEG_CH, DEG_CH)]],
                        add=True)
        return carry

    lax.fori_loop(0, EPT_DEG // DEG_CH, body, 0)
    plsc.subcore_barrier()

    @pl.when(c == 0)
    def _():
        pltpu.sync_copy(deg_sh.at[pl.ds(s * ZPT, ZPT)],
                        deg0_hbm.at[pl.ds(s * ZPT, ZPT)])

    @pl.when(c == 1)
    def _():
        pltpu.sync_copy(deg_sh.at[pl.ds(s * ZPT, ZPT)],
                        deg1_hbm.at[pl.ds(s * ZPT, ZPT)])


@functools.partial(
    pl.kernel,
    out_type=(jax.ShapeDtypeStruct((N, DH), jnp.float32),
              jax.ShapeDtypeStruct((N, DH), jnp.float32)),
    mesh=_sc_mesh,
    scratch_types=[
        pltpu.VMEM((EPT,), jnp.int32),
        pltpu.VMEM((EPT,), jnp.int32),
        pltpu.VMEM((AGG_CH, DH), jnp.float32),
        pltpu.VMEM((AGG_CH, DH), jnp.float32),
        pltpu.VMEM_SHARED((N, DH), jnp.float32),
        pltpu.SemaphoreType.DMA,
        pltpu.SemaphoreType.DMA,
    ],
)
def _agg_kernel(src_hbm, dst_hbm, hs0_hbm, hs1_hbm, out0_hbm, out1_hbm,
                srcv, dstv, rows_a, rows_b, acc_sh, sem_a, sem_b):
    c = lax.axis_index("c")
    s = lax.axis_index("s")

    def run(hs_hbm, out_hbm):
        @pl.when(s < NS - 1)
        def _():
            pltpu.sync_copy(hs_hbm.at[pl.ds(s * RPT, RPT)],
                            acc_sh.at[pl.ds(s * RPT, RPT)])

        @pl.when(s == NS - 1)
        def _():
            pltpu.sync_copy(hs_hbm.at[pl.ds((NS - 1) * RPT, N - (NS - 1) * RPT)],
                            acc_sh.at[pl.ds((NS - 1) * RPT, N - (NS - 1) * RPT)])

        pltpu.sync_copy(src_hbm.at[pl.ds(s * EPT, EPT)], srcv)
        pltpu.sync_copy(dst_hbm.at[pl.ds(s * EPT, EPT)], dstv)
        plsc.subcore_barrier()

        def gather(j, buf, sem):
            pltpu.async_copy(hs_hbm.at[srcv.at[pl.ds(j * AGG_CH, AGG_CH)]],
                             buf, sem)

        def drain(buf, sem):
            pltpu.make_async_copy(hs_hbm.at[pl.ds(0, AGG_CH)], buf, sem).wait()

        def scatter(j, buf):
            pltpu.sync_copy(buf,
                            acc_sh.at[dstv.at[pl.ds(j * AGG_CH, AGG_CH)]],
                            add=True)

        gather(0, rows_a, sem_a)

        def body(i, carry):
            gather(2 * i + 1, rows_b, sem_b)
            drain(rows_a, sem_a)
            scatter(2 * i, rows_a)

            @pl.when(i < AGG_NCH // 2 - 1)
            def _():
                gather(2 * i + 2, rows_a, sem_a)

            drain(rows_b, sem_b)
            scatter(2 * i + 1, rows_b)
            return carry

        lax.fori_loop(0, AGG_NCH // 2, body, 0)
        plsc.subcore_barrier()

        @pl.when(s < NS - 1)
        def _():
            pltpu.sync_copy(acc_sh.at[pl.ds(s * RPT, RPT)],
                            out_hbm.at[pl.ds(s * RPT, RPT)])

        @pl.when(s == NS - 1)
        def _():
            pltpu.sync_copy(acc_sh.at[pl.ds((NS - 1) * RPT, N - (NS - 1) * RPT)],
                            out_hbm.at[pl.ds((NS - 1) * RPT, N - (NS - 1) * RPT)])

    @pl.when(c == 0)
    def _():
        run(hs0_hbm, out0_hbm)

    @pl.when(c == 1)
    def _():
        run(hs1_hbm, out1_hbm)


def _mm1_body(feat_ref, w_ref, d0_ref, d1_ref, hs0_ref, hs1_ref, dinv_ref):
    dinv = lax.rsqrt(d0_ref[...] + d1_ref[...] + 1.0)
    h = jnp.dot(feat_ref[...], w_ref[...],
                preferred_element_type=jnp.float32)
    hs = h * dinv
    hs0_ref[...] = hs[:, :DH]
    hs1_ref[...] = hs[:, DH:]
    dinv_ref[...] = dinv


_mm1 = pl.pallas_call(
    _mm1_body,
    grid=(N // BM,),
    in_specs=[
        pl.BlockSpec((BM, D), lambda i: (i, 0)),
        pl.BlockSpec((D, D), lambda i: (0, 0)),
        pl.BlockSpec((BM, 1), lambda i: (i, 0)),
        pl.BlockSpec((BM, 1), lambda i: (i, 0)),
    ],
    out_specs=[
        pl.BlockSpec((BM, DH), lambda i: (i, 0)),
        pl.BlockSpec((BM, DH), lambda i: (i, 0)),
        pl.BlockSpec((BM, 1), lambda i: (i, 0)),
    ],
    out_shape=[
        jax.ShapeDtypeStruct((N, DH), jnp.float32),
        jax.ShapeDtypeStruct((N, DH), jnp.float32),
        jax.ShapeDtypeStruct((N, 1), jnp.float32),
    ],
)


def _mm2_body(a0_ref, a1_ref, dinv_ref, b_ref, w_ref, o0_ref, o1_ref):
    dinv = dinv_ref[...]
    x = jnp.concatenate([a0_ref[...], a1_ref[...]], axis=1) * dinv + b_ref[...]
    x = jnp.maximum(x, 0.0)
    h = jnp.dot(x, w_ref[...], preferred_element_type=jnp.float32)
    hs = h * dinv
    o0_ref[...] = hs[:, :DH]
    o1_ref[...] = hs[:, DH:]


_mm2 = pl.pallas_call(
    _mm2_body,
    grid=(N // BM,),
    in_specs=[
        pl.BlockSpec((BM, DH), lambda i: (i, 0)),
        pl.BlockSpec((BM, DH), lambda i: (i, 0)),
        pl.BlockSpec((BM, 1), lambda i: (i, 0)),
        pl.BlockSpec((1, D), lambda i: (0, 0)),
        pl.BlockSpec((D, D), lambda i: (0, 0)),
    ],
    out_specs=[
        pl.BlockSpec((BM, DH), lambda i: (i, 0)),
        pl.BlockSpec((BM, DH), lambda i: (i, 0)),
    ],
    out_shape=[
        jax.ShapeDtypeStruct((N, DH), jnp.float32),
        jax.ShapeDtypeStruct((N, DH), jnp.float32),
    ],
)


def _tail_body(a0_ref, a1_ref, dinv_ref, b2_ref, w0_ref, b0_ref,
               wf1_ref, bf1_ref, wf2_ref, bf2_ref,
               emb_ref, log_ref, gmax_ref):
    i = pl.program_id(0)
    x = jnp.concatenate([a0_ref[...], a1_ref[...]], axis=1) * dinv_ref[...] \
        + b2_ref[...]
    x = jnp.maximum(x, 0.0)
    m = jnp.max(x, axis=0, keepdims=True)

    @pl.when(i == 0)
    def _():
        gmax_ref[...] = m

    @pl.when(i > 0)
    def _():
        gmax_ref[...] = jnp.maximum(gmax_ref[...], m)

    @pl.when(i == pl.num_programs(0) - 1)
    def _():
        g = gmax_ref[...]
        emb = jnp.dot(g, w0_ref[...], preferred_element_type=jnp.float32) \
            + b0_ref[...]
        emb_ref[...] = emb
        hh = jnp.maximum(
            jnp.dot(emb, wf1_ref[...], preferred_element_type=jnp.float32)
            + bf1_ref[...], 0.0)
        log_ref[...] = jnp.dot(hh, wf2_ref[...],
                               preferred_element_type=jnp.float32) + bf2_ref[...]


_tail = pl.pallas_call(
    _tail_body,
    grid=(N // BM,),
    in_specs=[
        pl.BlockSpec((BM, DH), lambda i: (i, 0)),
        pl.BlockSpec((BM, DH), lambda i: (i, 0)),
        pl.BlockSpec((BM, 1), lambda i: (i, 0)),
        pl.BlockSpec((1, D), lambda i: (0, 0)),
        pl.BlockSpec((D, DH), lambda i: (0, 0)),
        pl.BlockSpec((1, DH), lambda i: (0, 0)),
        pl.BlockSpec((DH, D), lambda i: (0, 0)),
        pl.BlockSpec((1, D), lambda i: (0, 0)),
        pl.BlockSpec((D, 10), lambda i: (0, 0)),
        pl.BlockSpec((1, 10), lambda i: (0, 0)),
    ],
    out_specs=[
        pl.BlockSpec((1, DH), lambda i: (0, 0)),
        pl.BlockSpec((1, 10), lambda i: (0, 0)),
    ],
    out_shape=[
        jax.ShapeDtypeStruct((1, DH), jnp.float32),
        jax.ShapeDtypeStruct((1, 10), jnp.float32),
    ],
    scratch_shapes=[pltpu.VMEM((1, D), jnp.float32)],
)


def kernel(features, edge_index, W1, b1, W2, b2, W0, b0, Wf1, bf1, Wf2, bf2):
    src = edge_index[0]
    dst = edge_index[1]

    d0p, d1p = _deg_kernel(dst)
    d0 = d0p[:N].reshape(N, 1)
    d1 = d1p[:N].reshape(N, 1)

    hs0, hs1, dinv = _mm1(features, W1, d0, d1)
    a0, a1 = _agg_kernel(src, dst, hs0, hs1)
    g0, g1 = _mm2(a0, a1, dinv, b1.reshape(1, D), W2)
    c0, c1 = _agg_kernel(src, dst, g0, g1)
    emb, logits = _tail(c0, c1, dinv, b2.reshape(1, D),
                        W0, b0.reshape(1, DH),
                        Wf1, bf1.reshape(1, D),
                        Wf2, bf2.reshape(1, 10))
    return emb.reshape(DH), logits.reshape(10)
